# baseline (speedup 1.0000x reference)
import functools

import jax
import jax.numpy as jnp
from jax import lax
from jax.experimental import pallas as pl
from jax.experimental.pallas import tpu as pltpu
from jax.experimental.pallas import tpu_sc as plsc

_N = 10000
_NP = 10112
_F = 128
_E = 320000
_K = 10
_NC = 2
_NS = 16
_NW = _NC * _NS
_C = 120
_NCH = 88
_EPW = _C * _NCH
_EPAD = _NW * _EPW
_RPT = _NP // _NS
_RB = 632
_GRID = _NP // _RB

_mesh = plsc.VectorSubcoreMesh(core_axis_name="c", subcore_axis_name="s")


@functools.partial(
    pl.kernel,
    out_type=jax.ShapeDtypeStruct((_NW, _NP), jnp.float32),
    mesh=_mesh,
    compiler_params=pltpu.CompilerParams(needs_layout_passes=False),
    scratch_types=[
        pltpu.VMEM((_E // _NW,), jnp.int32),
        pltpu.VMEM((_NP,), jnp.float32),
    ],
)
def _sc_degree(dst_hbm, deg_out, dstv, degv):
    wid = lax.axis_index("s") * _NC + lax.axis_index("c")
    pltpu.sync_copy(dst_hbm.at[wid], dstv)

    def _zero(i, _):
        degv[pl.ds(i * 16, 16)] = jnp.zeros((16,), jnp.float32)
        return ()

    lax.fori_loop(0, _NP // 16, _zero, ())

    ones = jnp.ones((16,), jnp.float32)

    def _acc(i, _):
        d = dstv[pl.ds(i * 16, 16)]
        plsc.addupdate_scatter(degv, [d], ones)
        return ()

    lax.fori_loop(0, (_E // _NW) // 16, _acc, ())
    pltpu.sync_copy(degv, deg_out.at[wid])


@functools.partial(
    pl.kernel,
    out_type=jax.ShapeDtypeStruct((_NC, _NP, _F), jnp.float32),
    mesh=_mesh,
    compiler_params=pltpu.CompilerParams(needs_layout_passes=False),
    scratch_types=[
        pltpu.VMEM((4, 2, _C), jnp.int32),
        pltpu.VMEM((3, _C, _F), jnp.float32),
        pltpu.VMEM_SHARED((_NP, _F), jnp.float32),
        [pltpu.SemaphoreType.DMA] * 3,
        [pltpu.SemaphoreType.DMA] * 3,
        [pltpu.SemaphoreType.DMA] * 4,
    ],
)
def _sc_edges(idx2_hbm, g_hbm, zeros_hbm, acc_out,
              idx2b, rows, shacc, gsems, ssems, dsems):
    cid = lax.axis_index("c")
    sid = lax.axis_index("s")
    wid = sid * _NC + cid
    stripe = pl.ds(sid * _RPT, _RPT)

    pltpu.sync_copy(zeros_hbm, shacc.at[stripe])

    def _fire_idx(c, si):
        pltpu.async_copy(idx2_hbm.at[wid, c], idx2b.at[si], dsems[si])

    def _wait_idx(si):
        pltpu.make_async_copy(idx2_hbm.at[wid, 0], idx2b.at[si], dsems[si]).wait()

    def _fire_g(c, sr, si):
        pltpu.async_copy(g_hbm.at[idx2b.at[si, 0]], rows.at[sr], gsems[sr])

    def _wait_g(sr):
        pltpu.make_async_copy(g_hbm.at[idx2b.at[0, 0]], rows.at[sr], gsems[sr]).wait()

    def _scatter(sr, si):
        pltpu.sync_copy(rows.at[sr], shacc.at[idx2b.at[si, 1]], add=True)

    def _wait_s(sr):
        pass

    _fire_idx(0, 0)
    _fire_idx(1, 1)
    plsc.subcore_barrier()
    _wait_idx(0)
    _fire_g(0, 0, 0)

    def _step(c, sr, si, first=False, last=False):
        _wait_g(sr)
        _scatter(sr, si)
        if not first:
            _wait_s((sr + 1) % 3)
        if not last:
            _wait_idx((si + 1) % 4)
            _fire_g(c + 1, (sr + 1) % 3, (si + 1) % 4)
            if not isinstance(c, int) or c + 2 <= _NCH - 1:
                _fire_idx(c + 2, (si + 2) % 4)

    _step(0, 0, 0, first=True)
    _step(1, 1, 1, first=True)

    def _block(i, _):
        for j in range(12):
            _step(2 + i * 12 + j, (2 + j) % 3, (2 + j) % 4)
        return ()

    lax.fori_loop(0, (_NCH - 4) // 12, _block, ())
    _step(_NCH - 2, (_NCH - 2) % 3, (_NCH - 2) % 4)
    _step(_NCH - 1, (_NCH - 1) % 3, (_NCH - 1) % 4, last=True)
    _wait_s((_NCH - 2) % 3)
    _wait_s((_NCH - 1) % 3)

    plsc.subcore_barrier()
    pltpu.sync_copy(shacc.at[stripe], acc_out.at[cid, stripe])


def _mlp_body(x_ref, w1_ref, b1_ref, w2_ref, b2_ref, degt_ref, t0_ref,
              hid_ref, g_ref, dinv_ref):
    h = jnp.maximum(jnp.dot(x_ref[...], w1_ref[...],
                            preferred_element_type=jnp.float32) + b1_ref[...], 0.0)
    h0 = jnp.dot(h, w2_ref[...], preferred_element_type=jnp.float32) + b2_ref[...]
    deg = 1.0 + jnp.sum(degt_ref[...], axis=1, keepdims=True)
    dinv = lax.rsqrt(deg)
    hid_ref[...] = t0_ref[0] * h0
    g_ref[...] = dinv * h0
    dinv_ref[...] = dinv


def _mlp_init(xp, W1, b1r, W2, b2r, degt, t0):
    return pl.pallas_call(
        _mlp_body,
        grid=(_GRID,),
        in_specs=[
            pl.BlockSpec((_RB, _F), lambda i: (i, 0)),
            pl.BlockSpec((_F, _F), lambda i: (0, 0)),
            pl.BlockSpec((1, _F), lambda i: (0, 0)),
            pl.BlockSpec((_F, _F), lambda i: (0, 0)),
            pl.BlockSpec((1, _F), lambda i: (0, 0)),
            pl.BlockSpec((_RB, _NW), lambda i: (i, 0)),
            pl.BlockSpec(memory_space=pltpu.SMEM),
        ],
        out_specs=[
            pl.BlockSpec((_RB, _F), lambda i: (i, 0)),
            pl.BlockSpec((_RB, _F), lambda i: (i, 0)),
            pl.BlockSpec((_RB, 1), lambda i: (i, 0)),
        ],
        out_shape=[
            jax.ShapeDtypeStruct((_NP, _F), jnp.float32),
            jax.ShapeDtypeStruct((_NP, _F), jnp.float32),
            jax.ShapeDtypeStruct((_NP, 1), jnp.float32),
        ],
    )(xp, W1, b1r, W2, b2r, degt, t0)


def _comb_body(acc_ref, g_ref, hid_ref, dinv_ref, tk_ref, hido_ref, go_ref):
    dinv = dinv_ref[...]
    h = dinv * (acc_ref[0] + acc_ref[1] + g_ref[...])
    hido_ref[...] = hid_ref[...] + tk_ref[0] * h
    go_ref[...] = dinv * h


def _combine(acc, g, hidden, dinv, tk):
    return pl.pallas_call(
        _comb_body,
        grid=(_GRID,),
        in_specs=[
            pl.BlockSpec((_NC, _RB, _F), lambda i: (0, i, 0)),
            pl.BlockSpec((_RB, _F), lambda i: (i, 0)),
            pl.BlockSpec((_RB, _F), lambda i: (i, 0)),
            pl.BlockSpec((_RB, 1), lambda i: (i, 0)),
            pl.BlockSpec(memory_space=pltpu.SMEM),
        ],
        out_specs=[
            pl.BlockSpec((_RB, _F), lambda i: (i, 0)),
            pl.BlockSpec((_RB, _F), lambda i: (i, 0)),
        ],
        out_shape=[
            jax.ShapeDtypeStruct((_NP, _F), jnp.float32),
            jax.ShapeDtypeStruct((_NP, _F), jnp.float32),
        ],
    )(acc, g, hidden, dinv, tk)


def kernel(x, edge_index, W1, b1, W2, b2, temp):
    src = edge_index[0]
    dst = edge_index[1]
    pad = _EPAD - _E
    srcp = jnp.concatenate([src, jnp.zeros((pad,), src.dtype)]).reshape(_NW, _NCH, _C)
    dstp = jnp.concatenate([dst, jnp.full((pad,), _N, dst.dtype)]).reshape(_NW, _NCH, _C)
    idx2 = jnp.stack([srcp, dstp], axis=2)
    xp = jnp.concatenate([x, jnp.zeros((_NP - _N, _F), x.dtype)])

    degp = _sc_degree(dst.reshape(_NW, _E // _NW))
    degt = degp.T

    hidden, g, dinv = _mlp_init(xp, W1, b1.reshape(1, _F), W2, b2.reshape(1, _F),
                                degt, temp[0:1])

    zeros = jnp.zeros((_RPT, _F), jnp.float32)
    for k in range(_K):
        acc = _sc_edges(idx2, g, zeros)
        hidden, g = _combine(acc, g, hidden, dinv, temp[k + 1:k + 2])
    return hidden[:_N]


# speedup vs baseline: 1.6710x; 1.6710x over previous
import functools

import jax
import jax.numpy as jnp
from jax import lax
from jax.experimental import pallas as pl
from jax.experimental.pallas import tpu as pltpu
from jax.experimental.pallas import tpu_sc as plsc

_N = 10000
_NP = 10112
_F = 128
_E = 320000
_K = 10
_NC = 2
_NS = 16
_NW = _NC * _NS
_C = 128
_NCH = 80
_EPW = _C * _NCH
_EPAD = _NW * _EPW
_RPT = _NP // _NS
_RB = 632
_GRID = _NP // _RB

_mesh = plsc.VectorSubcoreMesh(core_axis_name="c", subcore_axis_name="s")


@functools.partial(
    pl.kernel,
    out_type=jax.ShapeDtypeStruct((_NW, _NP), jnp.float32),
    mesh=_mesh,
    compiler_params=pltpu.CompilerParams(needs_layout_passes=False),
    scratch_types=[
        pltpu.VMEM((_E // _NW,), jnp.int32),
        pltpu.VMEM((_NP,), jnp.float32),
    ],
)
def _sc_degree(dst_hbm, deg_out, dstv, degv):
    wid = lax.axis_index("s") * _NC + lax.axis_index("c")
    pltpu.sync_copy(dst_hbm.at[wid], dstv)

    def _zero(i, _):
        degv[pl.ds(i * 16, 16)] = jnp.zeros((16,), jnp.float32)
        return ()

    lax.fori_loop(0, _NP // 16, _zero, ())

    ones = jnp.ones((16,), jnp.float32)

    def _acc(i, _):
        d = dstv[pl.ds(i * 16, 16)]
        plsc.addupdate_scatter(degv, [d], ones)
        return ()

    lax.fori_loop(0, (_E // _NW) // 16, _acc, ())
    pltpu.sync_copy(degv, deg_out.at[wid])


@functools.partial(
    pl.kernel,
    out_type=jax.ShapeDtypeStruct((_NC, _NP, _F), jnp.float32),
    mesh=_mesh,
    compiler_params=pltpu.CompilerParams(needs_layout_passes=False),
    scratch_types=[
        pltpu.VMEM((_NCH, _C), jnp.int32),
        pltpu.VMEM((2, _C), jnp.int32),
        pltpu.VMEM((2, _C, _F), jnp.float32),
        pltpu.VMEM_SHARED((_NP, _F), jnp.float32),
        pltpu.SemaphoreType.DMA,
        pltpu.SemaphoreType.DMA,
        pltpu.SemaphoreType.DMA,
        pltpu.SemaphoreType.DMA,
    ],
)
def _sc_edges(srcp_hbm, dstp_hbm, g_hbm, zeros_hbm, acc_out,
              idxs, idxd, rows, shacc, gsem0, gsem1, dsem0, dsem1):
    cid = lax.axis_index("c")
    sid = lax.axis_index("s")
    wid = sid * _NC + cid
    stripe = pl.ds(sid * _RPT, _RPT)
    gsems = (gsem0, gsem1)
    dsems = (dsem0, dsem1)

    pltpu.sync_copy(srcp_hbm.at[wid], idxs)
    for b in range(2):
        pltpu.async_copy(dstp_hbm.at[wid, b], idxd.at[b], dsems[b])
        pltpu.async_copy(g_hbm.at[idxs.at[b]], rows.at[b], gsems[b])
    pltpu.sync_copy(zeros_hbm, shacc.at[stripe])
    plsc.subcore_barrier()

    def _step(b):
        pltpu.make_async_copy(dstp_hbm.at[wid, 0], idxd.at[b], dsems[b]).wait()
        pltpu.make_async_copy(g_hbm.at[idxs.at[0]], rows.at[b], gsems[b]).wait()
        pltpu.sync_copy(rows.at[b], shacc.at[idxd.at[b]], add=True)

    def _round(c2, _):
        for b in range(2):
            c = c2 * 2 + b
            _step(b)
            pltpu.async_copy(dstp_hbm.at[wid, c + 2], idxd.at[b], dsems[b])
            pltpu.async_copy(g_hbm.at[idxs.at[c + 2]], rows.at[b], gsems[b])
        return ()

    lax.fori_loop(0, _NCH // 2 - 1, _round, ())
    for b in range(2):
        _step(b)

    plsc.subcore_barrier()
    pltpu.sync_copy(shacc.at[stripe], acc_out.at[cid, stripe])


def _mlp_body(x_ref, w1_ref, b1_ref, w2_ref, b2_ref, degt_ref, t0_ref,
              hid_ref, g_ref, dinv_ref):
    h = jnp.maximum(jnp.dot(x_ref[...], w1_ref[...],
                            preferred_element_type=jnp.float32) + b1_ref[...], 0.0)
    h0 = jnp.dot(h, w2_ref[...], preferred_element_type=jnp.float32) + b2_ref[...]
    deg = 1.0 + jnp.sum(degt_ref[...], axis=1, keepdims=True)
    dinv = lax.rsqrt(deg)
    hid_ref[...] = t0_ref[0] * h0
    g_ref[...] = dinv * h0
    dinv_ref[...] = dinv


def _mlp_init(xp, W1, b1r, W2, b2r, degt, t0):
    return pl.pallas_call(
        _mlp_body,
        grid=(_GRID,),
        in_specs=[
            pl.BlockSpec((_RB, _F), lambda i: (i, 0)),
            pl.BlockSpec((_F, _F), lambda i: (0, 0)),
            pl.BlockSpec((1, _F), lambda i: (0, 0)),
            pl.BlockSpec((_F, _F), lambda i: (0, 0)),
            pl.BlockSpec((1, _F), lambda i: (0, 0)),
            pl.BlockSpec((_RB, _NW), lambda i: (i, 0)),
            pl.BlockSpec(memory_space=pltpu.SMEM),
        ],
        out_specs=[
            pl.BlockSpec((_RB, _F), lambda i: (i, 0)),
            pl.BlockSpec((_RB, _F), lambda i: (i, 0)),
            pl.BlockSpec((_RB, 1), lambda i: (i, 0)),
        ],
        out_shape=[
            jax.ShapeDtypeStruct((_NP, _F), jnp.float32),
            jax.ShapeDtypeStruct((_NP, _F), jnp.float32),
            jax.ShapeDtypeStruct((_NP, 1), jnp.float32),
        ],
    )(xp, W1, b1r, W2, b2r, degt, t0)


def _comb_body(acc_ref, g_ref, hid_ref, dinv_ref, tk_ref, hido_ref, go_ref):
    dinv = dinv_ref[...]
    h = dinv * (acc_ref[0] + acc_ref[1] + g_ref[...])
    hido_ref[...] = hid_ref[...] + tk_ref[0] * h
    go_ref[...] = dinv * h


def _combine(acc, g, hidden, dinv, tk):
    return pl.pallas_call(
        _comb_body,
        grid=(_GRID,),
        in_specs=[
            pl.BlockSpec((_NC, _RB, _F), lambda i: (0, i, 0)),
            pl.BlockSpec((_RB, _F), lambda i: (i, 0)),
            pl.BlockSpec((_RB, _F), lambda i: (i, 0)),
            pl.BlockSpec((_RB, 1), lambda i: (i, 0)),
            pl.BlockSpec(memory_space=pltpu.SMEM),
        ],
        out_specs=[
            pl.BlockSpec((_RB, _F), lambda i: (i, 0)),
            pl.BlockSpec((_RB, _F), lambda i: (i, 0)),
        ],
        out_shape=[
            jax.ShapeDtypeStruct((_NP, _F), jnp.float32),
            jax.ShapeDtypeStruct((_NP, _F), jnp.float32),
        ],
    )(acc, g, hidden, dinv, tk)


def kernel(x, edge_index, W1, b1, W2, b2, temp):
    src = edge_index[0]
    dst = edge_index[1]
    pad = _EPAD - _E
    srcp = jnp.concatenate([src, jnp.zeros((pad,), src.dtype)]).reshape(_NW, _NCH, _C)
    dstp = jnp.concatenate([dst, jnp.full((pad,), _N, dst.dtype)]).reshape(_NW, _NCH, _C)
    xp = jnp.concatenate([x, jnp.zeros((_NP - _N, _F), x.dtype)])

    degp = _sc_degree(dst.reshape(_NW, _E // _NW))
    degt = degp.T

    hidden, g, dinv = _mlp_init(xp, W1, b1.reshape(1, _F), W2, b2.reshape(1, _F),
                                degt, temp[0:1])

    zeros = jnp.zeros((_RPT, _F), jnp.float32)
    for k in range(_K):
        acc = _sc_edges(srcp, dstp, g, zeros)
        hidden, g = _combine(acc, g, hidden, dinv, temp[k + 1:k + 2])
    return hidden[:_N]


# speedup vs baseline: 1.7392x; 1.0408x over previous
import functools

import jax
import jax.numpy as jnp
from jax import lax
from jax.experimental import pallas as pl
from jax.experimental.pallas import tpu as pltpu
from jax.experimental.pallas import tpu_sc as plsc

_N = 10000
_NP = 10240
_F = 128
_E = 320000
_K = 10
_NC = 2
_NS = 16
_NW = _NC * _NS
_C = 128
_NCH = 80
_EPW = _C * _NCH
_EPAD = _NW * _EPW
_RPT = _NP // _NS
_RB = 1024
_GRID = _NP // _RB

_mesh = plsc.VectorSubcoreMesh(core_axis_name="c", subcore_axis_name="s")


@functools.partial(
    pl.kernel,
    out_type=jax.ShapeDtypeStruct((_NW, _NP), jnp.float32),
    mesh=_mesh,
    compiler_params=pltpu.CompilerParams(needs_layout_passes=False),
    scratch_types=[
        pltpu.VMEM((_E // _NW,), jnp.int32),
        pltpu.VMEM((_NP,), jnp.float32),
    ],
)
def _sc_degree(dst_hbm, deg_out, dstv, degv):
    wid = lax.axis_index("s") * _NC + lax.axis_index("c")
    pltpu.sync_copy(dst_hbm.at[wid], dstv)

    def _zero(i, _):
        degv[pl.ds(i * 16, 16)] = jnp.zeros((16,), jnp.float32)
        return ()

    lax.fori_loop(0, _NP // 16, _zero, ())

    ones = jnp.ones((16,), jnp.float32)

    def _acc(i, _):
        d = dstv[pl.ds(i * 16, 16)]
        plsc.addupdate_scatter(degv, [d], ones)
        return ()

    lax.fori_loop(0, (_E // _NW) // 16, _acc, ())
    pltpu.sync_copy(degv, deg_out.at[wid])


@functools.partial(
    pl.kernel,
    out_type=jax.ShapeDtypeStruct((_NC, _NP, _F), jnp.float32),
    mesh=_mesh,
    compiler_params=pltpu.CompilerParams(needs_layout_passes=False),
    scratch_types=[
        pltpu.VMEM((_NCH, _C), jnp.int32),
        pltpu.VMEM((2, _C), jnp.int32),
        pltpu.VMEM((2, _C, _F), jnp.float32),
        pltpu.VMEM_SHARED((_NP, _F), jnp.float32),
        pltpu.SemaphoreType.DMA,
        pltpu.SemaphoreType.DMA,
        pltpu.SemaphoreType.DMA,
        pltpu.SemaphoreType.DMA,
    ],
)
def _sc_edges(srcp_hbm, dstp_hbm, g_hbm, zeros_hbm, acc_out,
              idxs, idxd, rows, shacc, gsem0, gsem1, dsem0, dsem1):
    cid = lax.axis_index("c")
    sid = lax.axis_index("s")
    wid = sid * _NC + cid
    stripe = pl.ds(sid * _RPT, _RPT)
    gsems = (gsem0, gsem1)
    dsems = (dsem0, dsem1)

    pltpu.sync_copy(srcp_hbm.at[wid], idxs)
    for b in range(2):
        pltpu.async_copy(dstp_hbm.at[wid, b], idxd.at[b], dsems[b])
        pltpu.async_copy(g_hbm.at[idxs.at[b]], rows.at[b], gsems[b])
    pltpu.sync_copy(zeros_hbm, shacc.at[stripe])
    plsc.subcore_barrier()

    def _step(b):
        pltpu.make_async_copy(dstp_hbm.at[wid, 0], idxd.at[b], dsems[b]).wait()
        pltpu.make_async_copy(g_hbm.at[idxs.at[0]], rows.at[b], gsems[b]).wait()
        pltpu.sync_copy(rows.at[b], shacc.at[idxd.at[b]], add=True)

    def _round(c2, _):
        for b in range(2):
            c = c2 * 2 + b
            _step(b)
            pltpu.async_copy(dstp_hbm.at[wid, c + 2], idxd.at[b], dsems[b])
            pltpu.async_copy(g_hbm.at[idxs.at[c + 2]], rows.at[b], gsems[b])
        return ()

    lax.fori_loop(0, _NCH // 2 - 1, _round, ())
    for b in range(2):
        _step(b)

    plsc.subcore_barrier()
    pltpu.sync_copy(shacc.at[stripe], acc_out.at[cid, stripe])


def _mlp_body(x_ref, w1_ref, b1_ref, w2_ref, b2_ref, degt_ref, t0_ref,
              hid_ref, g_ref, dinv_ref):
    h = jnp.maximum(jnp.dot(x_ref[...], w1_ref[...],
                            preferred_element_type=jnp.float32) + b1_ref[...], 0.0)
    h0 = jnp.dot(h, w2_ref[...], preferred_element_type=jnp.float32) + b2_ref[...]
    deg = 1.0 + jnp.sum(degt_ref[...], axis=1, keepdims=True)
    dinv = lax.rsqrt(deg)
    hid_ref[...] = t0_ref[0] * h0
    g_ref[...] = dinv * h0
    dinv_ref[...] = dinv


def _mlp_init(xp, W1, b1r, W2, b2r, degt, t0):
    return pl.pallas_call(
        _mlp_body,
        grid=(_GRID,),
        in_specs=[
            pl.BlockSpec((_RB, _F), lambda i: (i, 0)),
            pl.BlockSpec((_F, _F), lambda i: (0, 0)),
            pl.BlockSpec((1, _F), lambda i: (0, 0)),
            pl.BlockSpec((_F, _F), lambda i: (0, 0)),
            pl.BlockSpec((1, _F), lambda i: (0, 0)),
            pl.BlockSpec((_RB, _NW), lambda i: (i, 0)),
            pl.BlockSpec(memory_space=pltpu.SMEM),
        ],
        out_specs=[
            pl.BlockSpec((_RB, _F), lambda i: (i, 0)),
            pl.BlockSpec((_RB, _F), lambda i: (i, 0)),
            pl.BlockSpec((_RB, 1), lambda i: (i, 0)),
        ],
        out_shape=[
            jax.ShapeDtypeStruct((_NP, _F), jnp.float32),
            jax.ShapeDtypeStruct((_NP, _F), jnp.float32),
            jax.ShapeDtypeStruct((_NP, 1), jnp.float32),
        ],
    )(xp, W1, b1r, W2, b2r, degt, t0)


def _comb_body(acc_ref, g_ref, hid_ref, dinv_ref, tk_ref, hido_ref, go_ref):
    dinv = dinv_ref[...]
    h = dinv * (acc_ref[0] + acc_ref[1] + g_ref[...])
    hido_ref[...] = hid_ref[...] + tk_ref[0] * h
    go_ref[...] = dinv * h


def _combine(acc, g, hidden, dinv, tk):
    return pl.pallas_call(
        _comb_body,
        grid=(_GRID,),
        in_specs=[
            pl.BlockSpec((_NC, _RB, _F), lambda i: (0, i, 0)),
            pl.BlockSpec((_RB, _F), lambda i: (i, 0)),
            pl.BlockSpec((_RB, _F), lambda i: (i, 0)),
            pl.BlockSpec((_RB, 1), lambda i: (i, 0)),
            pl.BlockSpec(memory_space=pltpu.SMEM),
        ],
        out_specs=[
            pl.BlockSpec((_RB, _F), lambda i: (i, 0)),
            pl.BlockSpec((_RB, _F), lambda i: (i, 0)),
        ],
        out_shape=[
            jax.ShapeDtypeStruct((_NP, _F), jnp.float32),
            jax.ShapeDtypeStruct((_NP, _F), jnp.float32),
        ],
    )(acc, g, hidden, dinv, tk)


def kernel(x, edge_index, W1, b1, W2, b2, temp):
    src = edge_index[0]
    dst = edge_index[1]
    pad = _EPAD - _E
    srcp = jnp.concatenate([src, jnp.zeros((pad,), src.dtype)]).reshape(_NW, _NCH, _C)
    dstp = jnp.concatenate([dst, jnp.full((pad,), _N, dst.dtype)]).reshape(_NW, _NCH, _C)
    xp = jnp.concatenate([x, jnp.zeros((_NP - _N, _F), x.dtype)])

    degp = _sc_degree(dst.reshape(_NW, _E // _NW))
    degt = degp.T

    hidden, g, dinv = _mlp_init(xp, W1, b1.reshape(1, _F), W2, b2.reshape(1, _F),
                                degt, temp[0:1])

    zeros = jnp.zeros((_RPT, _F), jnp.float32)
    for k in range(_K):
        acc = _sc_edges(srcp, dstp, g, zeros)
        hidden, g = _combine(acc, g, hidden, dinv, temp[k + 1:k + 2])
    return hidden[:_N]


# speedup vs baseline: 1.8658x; 1.0728x over previous
import functools

import jax
import jax.numpy as jnp
from jax import lax
from jax.experimental import pallas as pl
from jax.experimental.pallas import tpu as pltpu
from jax.experimental.pallas import tpu_sc as plsc

_N = 10000
_NP = 10240
_F = 128
_E = 320000
_K = 10
_NC = 2
_NS = 16
_NW = _NC * _NS
_C = 128
_NCH = 80
_EPW = _C * _NCH
_EPAD = _NW * _EPW
_RPT = _NP // _NS
_RB = 1024
_GRID = _NP // _RB

_mesh = plsc.VectorSubcoreMesh(core_axis_name="c", subcore_axis_name="s")


@functools.partial(
    pl.kernel,
    out_type=jax.ShapeDtypeStruct((_NW, _NP), jnp.float32),
    mesh=_mesh,
    compiler_params=pltpu.CompilerParams(needs_layout_passes=False),
    scratch_types=[
        pltpu.VMEM((_E // _NW,), jnp.int32),
        pltpu.VMEM((_NP,), jnp.float32),
    ],
)
def _sc_degree(dst_hbm, deg_out, dstv, degv):
    wid = lax.axis_index("s") * _NC + lax.axis_index("c")
    pltpu.sync_copy(dst_hbm.at[wid], dstv)

    def _zero(i, _):
        degv[pl.ds(i * 16, 16)] = jnp.zeros((16,), jnp.float32)
        return ()

    lax.fori_loop(0, _NP // 16, _zero, ())

    ones = jnp.ones((16,), jnp.float32)

    def _acc(i, _):
        d = dstv[pl.ds(i * 16, 16)]
        plsc.addupdate_scatter(degv, [d], ones)
        return ()

    lax.fori_loop(0, (_E // _NW) // 16, _acc, ())
    pltpu.sync_copy(degv, deg_out.at[wid])


@functools.partial(
    pl.kernel,
    out_type=jax.ShapeDtypeStruct((_NC, _NP, _F), jnp.float32),
    mesh=_mesh,
    compiler_params=pltpu.CompilerParams(needs_layout_passes=False),
    scratch_types=[
        pltpu.VMEM((_NCH, _C), jnp.int32),
        pltpu.VMEM((2, _C), jnp.int32),
        pltpu.VMEM((2, _C, _F), jnp.float32),
        pltpu.VMEM_SHARED((_NP, _F), jnp.float32),
        pltpu.SemaphoreType.DMA,
        pltpu.SemaphoreType.DMA,
        pltpu.SemaphoreType.DMA,
        pltpu.SemaphoreType.DMA,
    ],
)
def _sc_edges(srcp_hbm, dstp_hbm, g_hbm, zeros_hbm, acc_out,
              idxs, idxd, rows, shacc, gsem0, gsem1, dsem0, dsem1):
    cid = lax.axis_index("c")
    sid = lax.axis_index("s")
    wid = sid * _NC + cid
    stripe = pl.ds(sid * _RPT, _RPT)
    gsems = (gsem0, gsem1)
    dsems = (dsem0, dsem1)

    pltpu.sync_copy(srcp_hbm.at[wid], idxs)
    for b in range(2):
        pltpu.async_copy(dstp_hbm.at[wid, b], idxd.at[b], dsems[b])
        pltpu.async_copy(g_hbm.at[idxs.at[b]], rows.at[b], gsems[b])
    pltpu.sync_copy(zeros_hbm, shacc.at[stripe])
    plsc.subcore_barrier()

    def _step(b):
        pltpu.make_async_copy(dstp_hbm.at[wid, 0], idxd.at[b], dsems[b]).wait()
        pltpu.make_async_copy(g_hbm.at[idxs.at[0]], rows.at[b], gsems[b]).wait()
        pltpu.sync_copy(rows.at[b], shacc.at[idxd.at[b]], add=True)

    def _round(c2, _):
        for b in range(2):
            c = c2 * 2 + b
            _step(b)
            pltpu.async_copy(dstp_hbm.at[wid, c + 2], idxd.at[b], dsems[b])
            pltpu.async_copy(g_hbm.at[idxs.at[c + 2]], rows.at[b], gsems[b])
        return ()

    lax.fori_loop(0, _NCH // 2 - 1, _round, ())
    for b in range(2):
        _step(b)

    plsc.subcore_barrier()
    pltpu.sync_copy(shacc.at[stripe], acc_out.at[cid, stripe])


def _mlp_body(x_ref, w1_ref, b1_ref, w2_ref, b2_ref, degt_ref, t0_ref,
              hid_ref, g_ref, dinv_ref):
    h = jnp.maximum(jnp.dot(x_ref[...], w1_ref[...],
                            preferred_element_type=jnp.float32) + b1_ref[...], 0.0)
    h0 = jnp.dot(h, w2_ref[...], preferred_element_type=jnp.float32) + b2_ref[...]
    deg = 1.0 + jnp.sum(degt_ref[...], axis=1, keepdims=True)
    dinv = lax.rsqrt(deg)
    hid_ref[...] = t0_ref[0] * h0
    g_ref[...] = dinv * h0
    dinv_ref[...] = dinv


def _mlp_init(xp, W1, b1r, W2, b2r, degt, t0):
    return pl.pallas_call(
        _mlp_body,
        grid=(_GRID,),
        in_specs=[
            pl.BlockSpec((_RB, _F), lambda i: (i, 0)),
            pl.BlockSpec((_F, _F), lambda i: (0, 0)),
            pl.BlockSpec((1, _F), lambda i: (0, 0)),
            pl.BlockSpec((_F, _F), lambda i: (0, 0)),
            pl.BlockSpec((1, _F), lambda i: (0, 0)),
            pl.BlockSpec((_RB, _NW), lambda i: (i, 0)),
            pl.BlockSpec(memory_space=pltpu.SMEM),
        ],
        out_specs=[
            pl.BlockSpec((_RB, _F), lambda i: (i, 0)),
            pl.BlockSpec((_RB, _F), lambda i: (i, 0)),
            pl.BlockSpec((_RB, 1), lambda i: (i, 0)),
        ],
        out_shape=[
            jax.ShapeDtypeStruct((_NP, _F), jnp.float32),
            jax.ShapeDtypeStruct((_NP, _F), jnp.float32),
            jax.ShapeDtypeStruct((_NP, 1), jnp.float32),
        ],
    )(xp, W1, b1r, W2, b2r, degt, t0)


def _comb_body(acc_ref, g_ref, hid_ref, dinv_ref, tk_ref, hido_ref, go_ref):
    dinv = dinv_ref[...]
    h = dinv * (acc_ref[0] + acc_ref[1] + g_ref[...])
    hido_ref[...] = hid_ref[...] + tk_ref[0] * h
    go_ref[...] = dinv * h


def _combine(acc, g, hidden, dinv, tk):
    return pl.pallas_call(
        _comb_body,
        grid=(_GRID,),
        input_output_aliases={1: 1, 2: 0},
        in_specs=[
            pl.BlockSpec((_NC, _RB, _F), lambda i: (0, i, 0)),
            pl.BlockSpec((_RB, _F), lambda i: (i, 0)),
            pl.BlockSpec((_RB, _F), lambda i: (i, 0)),
            pl.BlockSpec((_RB, 1), lambda i: (i, 0)),
            pl.BlockSpec(memory_space=pltpu.SMEM),
        ],
        out_specs=[
            pl.BlockSpec((_RB, _F), lambda i: (i, 0)),
            pl.BlockSpec((_RB, _F), lambda i: (i, 0)),
        ],
        out_shape=[
            jax.ShapeDtypeStruct((_NP, _F), jnp.float32),
            jax.ShapeDtypeStruct((_NP, _F), jnp.float32),
        ],
    )(acc, g, hidden, dinv, tk)


def kernel(x, edge_index, W1, b1, W2, b2, temp):
    src = edge_index[0]
    dst = edge_index[1]
    pad = _EPAD - _E
    srcp = jnp.concatenate([src, jnp.zeros((pad,), src.dtype)]).reshape(_NW, _NCH, _C)
    dstp = jnp.concatenate([dst, jnp.full((pad,), _N, dst.dtype)]).reshape(_NW, _NCH, _C)
    xp = jnp.concatenate([x, jnp.zeros((_NP - _N, _F), x.dtype)])

    degp = _sc_degree(dst.reshape(_NW, _E // _NW))
    degt = degp.T

    hidden, g, dinv = _mlp_init(xp, W1, b1.reshape(1, _F), W2, b2.reshape(1, _F),
                                degt, temp[0:1])

    zeros = jnp.zeros((_RPT, _F), jnp.float32)
    for k in range(_K):
        acc = _sc_edges(srcp, dstp, g, zeros)
        hidden, g = _combine(acc, g, hidden, dinv, temp[k + 1:k + 2])
    return hidden[:_N]
